# baseline (speedup 1.0000x reference)
import jax
import jax.numpy as jnp
from jax import lax
from jax.experimental import pallas as pl
from jax.experimental.pallas import tpu as pltpu
from jax.experimental.pallas import tpu_sc as plsc

VOCAB = 1000000
D_MODEL = 64
BATCH = 4096
SEQ = 200

NC = 2
NS = 16
NW = NC * NS

TCOLS = VOCAB // 128
TAIL = VOCAB - TCOLS * 128

_MESH = dict(core_axis_name="c", subcore_axis_name="s")


def _wid():
    return lax.axis_index("s") * NC + lax.axis_index("c")


def _repack_body(wt_hbm, tail_hbm, tbl_hbm, src, dst):
    wid = _wid()
    iota = lax.iota(jnp.int32, 16)
    rows_half = lax.shift_right_logical(iota, 1)
    colpar = (iota & 1) * 64

    def transpose_cols(n_l0):
        @pl.loop(0, D_MODEL)
        def _(d):
            cols = colpar + d
            for l0 in range(n_l0):
                v = src[d, pl.ds(l0 * 16, 16)]
                plsc.store_scatter(dst, [rows_half + (l0 * 8), cols], v)

    ncols = (TCOLS - wid + NW - 1) // NW
    @pl.loop(0, ncols)
    def _(m):
        k = wid + NW * m
        pltpu.sync_copy(wt_hbm.at[:, pl.ds(k * 128, 128)], src)
        transpose_cols(8)
        pltpu.sync_copy(dst, tbl_hbm.at[pl.ds(k * 64, 64)])

    @pl.when(wid == NW - 1)
    def _():
        pltpu.sync_copy(tail_hbm, dst.at[pl.ds(0, 32)])
        pltpu.sync_copy(
            dst.at[pl.ds(0, 32)], tbl_hbm.at[pl.ds(TCOLS * 64, 32)]
        )


def _repack(wt, tail2d):
    run = pl.kernel(
        _repack_body,
        out_type=jax.ShapeDtypeStruct((VOCAB // 2, 128), jnp.float32),
        mesh=plsc.VectorSubcoreMesh(**_MESH),
        scratch_types=[
            pltpu.VMEM((D_MODEL, 128), jnp.float32),
            pltpu.VMEM((D_MODEL, 128), jnp.float32),
        ],
        compiler_params=pltpu.CompilerParams(use_tc_tiling_on_sc=True, needs_layout_passes=False),
    )
    return run(wt, tail2d)


UNITS_PER_W = (SEQ // 8) * 32 // NW


def _lookup_body(tokt_hbm, tbl_hbm, out_hbm, idx_v, gath, trans, sem0, sem1):
    wid = _wid()
    iota = lax.iota(jnp.int32, 16)
    rows16 = [mm * 16 + iota for mm in range(4)]
    sems = [sem0, sem1]

    @pl.loop(0, UNITS_PER_W)
    def _(n):
        g = wid * UNITS_PER_W + n
        sblk = g // 32
        j = g - 32 * sblk
        s0 = sblk * 8
        pltpu.sync_copy(
            tokt_hbm.at[pl.ds(s0, 8), pl.ds(j * 128, 128)], idx_v
        )

        def fire(r8, b):
            pltpu.async_copy(tbl_hbm.at[idx_v.at[r8]], gath.at[b], sems[b])

        fire(0, 0)
        for r8 in range(8):
            b = r8 % 2
            if r8 < 7:
                fire(r8 + 1, 1 - b)
            pltpu.make_async_copy(
                tbl_hbm.at[idx_v.at[r8]], gath.at[b], sems[b]
            ).wait()
            @pl.loop(0, 128)
            def _(t):
                colt = jnp.broadcast_to(t, (16,)).astype(jnp.int32)
                for mm in range(4):
                    v = gath[b, t, pl.ds(mm * 16, 16)]
                    plsc.store_scatter(trans, [rows16[mm], colt], v)
            for i in range(8):
                pltpu.sync_copy(
                    trans.at[pl.ds(8 * i, 8)], out_hbm.at[s0 + r8, i, j]
                )


def _lookup(tokt, tbl):
    run = pl.kernel(
        _lookup_body,
        out_type=jax.ShapeDtypeStruct((SEQ, 8, 32, 8, 128), jnp.float32),
        mesh=plsc.VectorSubcoreMesh(**_MESH),
        scratch_types=[
            pltpu.VMEM((8, 128), jnp.int32),
            pltpu.VMEM((2, 128, D_MODEL), jnp.float32),
            pltpu.VMEM((D_MODEL, 128), jnp.float32),
            pltpu.SemaphoreType.DMA,
            pltpu.SemaphoreType.DMA,
        ],
        compiler_params=pltpu.CompilerParams(use_tc_tiling_on_sc=False, needs_layout_passes=False),
    )
    return run(tokt, tbl)


def kernel(token_ids, weight):
    wt = weight.T
    tail2d = weight[TCOLS * 128:].reshape(32, 128)
    tbl = _repack(wt, tail2d).reshape(VOCAB, D_MODEL)
    tokt = token_ids.T.astype(jnp.int32)
    out5 = _lookup(tokt, tbl)
    return out5.transpose(2, 4, 0, 1, 3).reshape(BATCH, SEQ, D_MODEL)


# speedup vs baseline: 1.4893x; 1.4893x over previous
import jax
import jax.numpy as jnp
from jax import lax
from jax.experimental import pallas as pl
from jax.experimental.pallas import tpu as pltpu
from jax.experimental.pallas import tpu_sc as plsc

VOCAB = 1000000
D_MODEL = 64
BATCH = 4096
SEQ = 200

NC = 2
NS = 16
NW = NC * NS

TCOLS = VOCAB // 128
TAIL = VOCAB - TCOLS * 128

_MESH = dict(core_axis_name="c", subcore_axis_name="s")


def _wid():
    return lax.axis_index("s") * NC + lax.axis_index("c")


def _repack_body(wt_hbm, tail_hbm, tbl_hbm, src, dst, isem, osem):
    wid = _wid()
    iota = lax.iota(jnp.int32, 16)
    rows_half = lax.shift_right_logical(iota, 1)
    colpar = (iota & 1) * 64

    ncols = (TCOLS - wid + NW - 1) // NW
    ntot = (TCOLS + NW - 1) // NW

    def fire_in(m, b):
        k = wid + NW * m
        pltpu.async_copy(wt_hbm.at[:, pl.ds(k * 128, 128)], src.at[b], isem)

    def wait_in(b):
        pltpu.make_async_copy(
            wt_hbm.at[:, pl.ds(0, 128)], src.at[b], isem
        ).wait()

    def fire_out(m, b):
        k = wid + NW * m
        pltpu.async_copy(dst.at[b], tbl_hbm.at[pl.ds(k * 64, 64)], osem)

    def wait_out(b):
        pltpu.make_async_copy(
            dst.at[b], tbl_hbm.at[pl.ds(0, 64)], osem
        ).wait()

    @pl.when(0 < ncols)
    def _():
        fire_in(0, 0)

    @pl.loop(0, ntot, step=2)
    def _(m0):
        for b in range(2):
            m = m0 + b

            @pl.when(m < ncols)
            def _():
                @pl.when(m + 1 < ncols)
                def _():
                    fire_in(m + 1, 1 - b)
                wait_in(b)

                @pl.when(m >= 2)
                def _():
                    wait_out(b)

                @pl.loop(0, D_MODEL, unroll=8)
                def _(d):
                    cols = colpar + d
                    for l0 in range(8):
                        v = src[b, d, pl.ds(l0 * 16, 16)]
                        plsc.store_scatter(
                            dst.at[b], [rows_half + (l0 * 8), cols], v
                        )
                fire_out(m, b)

    @pl.when(ncols >= 2)
    def _():
        wait_out(0)
        wait_out(1)

    @pl.when(wid == NW - 1)
    def _():
        pltpu.sync_copy(tail_hbm, src.at[0, pl.ds(0, 32)])
        pltpu.sync_copy(
            src.at[0, pl.ds(0, 32)], tbl_hbm.at[pl.ds(TCOLS * 64, 32)]
        )


def _repack(wt, tail2d):
    run = pl.kernel(
        _repack_body,
        out_type=jax.ShapeDtypeStruct((VOCAB // 2, 128), jnp.float32),
        mesh=plsc.VectorSubcoreMesh(**_MESH),
        scratch_types=[
            pltpu.VMEM((2, D_MODEL, 128), jnp.float32),
            pltpu.VMEM((2, D_MODEL, 128), jnp.float32),
            pltpu.SemaphoreType.DMA,
            pltpu.SemaphoreType.DMA,
        ],
        compiler_params=pltpu.CompilerParams(
            use_tc_tiling_on_sc=True, needs_layout_passes=False
        ),
    )
    return run(wt, tail2d)


UNITS_PER_W = (SEQ // 8) * 32 // NW


def _lookup_body(
    tokt_hbm, tbl_hbm, out_hbm, idx_v, gath, trans, xsem, gsem0, gsem1, osem
):
    wid = _wid()
    iota = lax.iota(jnp.int32, 16)
    ivecs = [lax.shift_right_logical(mm * 16 + iota, 3) for mm in range(4)]
    rvecs = [(mm * 16 + iota) & 7 for mm in range(4)]
    gsems = [gsem0, gsem1]

    def wait_wb(tb):
        pltpu.make_async_copy(
            trans.at[tb, :, :, pl.ds(0, 128)], out_hbm.at[0, :, 0], osem
        ).wait()

    def fire_idx(n, ib):
        g = wid * UNITS_PER_W + n
        sblk = g // 32
        j = g - 32 * sblk
        pltpu.async_copy(
            tokt_hbm.at[pl.ds(sblk * 8, 8), pl.ds(j * 128, 128)],
            idx_v.at[ib],
            xsem,
        )

    def wait_idx(ib):
        pltpu.make_async_copy(
            tokt_hbm.at[pl.ds(0, 8), pl.ds(0, 128)], idx_v.at[ib], xsem
        ).wait()

    fire_idx(0, 0)

    @pl.loop(0, UNITS_PER_W, step=2)
    def _(n0):
        for ib in range(2):
            n = n0 + ib
            g = wid * UNITS_PER_W + n
            sblk = g // 32
            j = g - 32 * sblk
            s0 = sblk * 8
            wait_idx(ib)

            @pl.when(n + 1 < UNITS_PER_W)
            def _():
                fire_idx(n + 1, 1 - ib)

            def fire_row(r8, b):
                pltpu.async_copy(
                    tbl_hbm.at[idx_v.at[ib, r8]], gath.at[b], gsems[b]
                )

            def wait_row(b):
                pltpu.make_async_copy(
                    tbl_hbm.at[idx_v.at[ib, 0]], gath.at[b], gsems[b]
                ).wait()

            fire_row(0, 0)
            for r8 in range(8):
                b = r8 % 2
                tb = r8 % 2
                if r8 < 7:
                    fire_row(r8 + 1, 1 - b)
                wait_row(b)
                if r8 >= 2:
                    wait_wb(tb)
                else:
                    @pl.when(n > 0)
                    def _():
                        wait_wb(tb)
                @pl.loop(0, 128, unroll=8)
                def _(t):
                    colt = jnp.broadcast_to(t, (16,)).astype(jnp.int32)
                    for mm in range(4):
                        v = gath[b, t, pl.ds(mm * 16, 16)]
                        plsc.store_scatter(
                            trans.at[tb], [ivecs[mm], rvecs[mm], colt], v
                        )
                pltpu.async_copy(
                    trans.at[tb, :, :, pl.ds(0, 128)],
                    out_hbm.at[s0 + r8, :, j],
                    osem,
                )

    wait_wb(0)
    wait_wb(1)


def _lookup(tokt, tbl):
    run = pl.kernel(
        _lookup_body,
        out_type=jax.ShapeDtypeStruct((SEQ, 8, 32, 8, 128), jnp.float32),
        mesh=plsc.VectorSubcoreMesh(**_MESH),
        scratch_types=[
            pltpu.VMEM((2, 8, 128), jnp.int32),
            pltpu.VMEM((2, 128, D_MODEL), jnp.float32),
            pltpu.VMEM((2, 8, 8, 131), jnp.float32),
            pltpu.SemaphoreType.DMA,
            pltpu.SemaphoreType.DMA,
            pltpu.SemaphoreType.DMA,
            pltpu.SemaphoreType.DMA,
        ],
        compiler_params=pltpu.CompilerParams(
            use_tc_tiling_on_sc=False, needs_layout_passes=False
        ),
    )
    return run(tokt, tbl)


N_BLOCKS = BATCH * SEQ // 128
BLOCKS_PER_W = N_BLOCKS // NW
K2 = 5
NBUF = 2
NCHUNK = BLOCKS_PER_W // K2


def _emb_body(tok_hbm, w_hbm, out_hbm, idx_v, rows_v, gsem):
    wid = _wid()
    base = wid * BLOCKS_PER_W
    pltpu.sync_copy(tok_hbm.at[pl.ds(base, BLOCKS_PER_W)], idx_v)

    def fire(c, b):
        for j in range(K2):
            pltpu.async_copy(
                w_hbm.at[idx_v.at[c * K2 + j]], rows_v.at[b, j], gsem
            )

    def drain(b):
        for j in range(K2):
            pltpu.make_async_copy(
                w_hbm.at[idx_v.at[j]], rows_v.at[b, j], gsem
            ).wait()

    for b in range(NBUF):
        fire(b, b)

    @pl.loop(0, NCHUNK - NBUF, step=NBUF)
    def _(c0):
        for b in range(NBUF):
            c = c0 + b
            drain(b)
            pltpu.sync_copy(rows_v.at[b], out_hbm.at[pl.ds(base + c * K2, K2)])
            fire(c + NBUF, b)

    for b in range(NBUF):
        c = NCHUNK - NBUF + b
        drain(b)
        pltpu.sync_copy(rows_v.at[b], out_hbm.at[pl.ds(base + c * K2, K2)])


def _emb_lookup(tok2d, tbl):
    run = pl.kernel(
        _emb_body,
        out_type=jax.ShapeDtypeStruct((N_BLOCKS, 128, D_MODEL), jnp.float32),
        mesh=plsc.VectorSubcoreMesh(**_MESH),
        scratch_types=[
            pltpu.VMEM((BLOCKS_PER_W, 128), jnp.int32),
            pltpu.VMEM((NBUF, K2, 128, D_MODEL), jnp.float32),
            pltpu.SemaphoreType.DMA,
        ],
        compiler_params=pltpu.CompilerParams(
            use_tc_tiling_on_sc=False, needs_layout_passes=False
        ),
    )
    return run(tok2d, tbl)


def kernel(token_ids, weight):
    wt = weight.T
    tail2d = weight[TCOLS * 128:].reshape(32, 128)
    tbl = _repack(wt, tail2d).reshape(VOCAB, D_MODEL)
    tok2d = token_ids.reshape(N_BLOCKS, 128).astype(jnp.int32)
    out = _emb_lookup(tok2d, tbl)
    return out.reshape(BATCH, SEQ, D_MODEL)


# speedup vs baseline: 1.7723x; 1.1900x over previous
import jax
import jax.numpy as jnp
from jax import lax
from jax.experimental import pallas as pl
from jax.experimental.pallas import tpu as pltpu
from jax.experimental.pallas import tpu_sc as plsc

VOCAB = 1000000
D_MODEL = 64
BATCH = 4096
SEQ = 200

NC = 2
NS = 16
NW = NC * NS

TCOLS = VOCAB // 128
TAIL = VOCAB - TCOLS * 128

_MESH = dict(core_axis_name="c", subcore_axis_name="s")


def _wid():
    return lax.axis_index("s") * NC + lax.axis_index("c")


def _repack_body(wt_hbm, tail_hbm, tbl_hbm, src, dst, isem, osem):
    wid = _wid()
    iota = lax.iota(jnp.int32, 16)
    rows_half = lax.shift_right_logical(iota, 1)
    colpar = (iota & 1) * 64

    ncols = (TCOLS - wid + NW - 1) // NW
    ntot = (TCOLS + NW - 1) // NW

    def fire_in(m, b):
        k = wid + NW * m
        pltpu.async_copy(wt_hbm.at[:, pl.ds(k * 128, 128)], src.at[b], isem)

    def wait_in(b):
        pltpu.make_async_copy(
            wt_hbm.at[:, pl.ds(0, 128)], src.at[b], isem
        ).wait()

    def fire_out(m, b):
        k = wid + NW * m
        pltpu.async_copy(dst.at[b], tbl_hbm.at[pl.ds(k * 64, 64)], osem)

    def wait_out(b):
        pltpu.make_async_copy(
            dst.at[b], tbl_hbm.at[pl.ds(0, 64)], osem
        ).wait()

    @pl.when(0 < ncols)
    def _():
        fire_in(0, 0)

    @pl.loop(0, ntot, step=2)
    def _(m0):
        for b in range(2):
            m = m0 + b

            @pl.when(m < ncols)
            def _():
                @pl.when(m + 1 < ncols)
                def _():
                    fire_in(m + 1, 1 - b)
                wait_in(b)

                @pl.when(m >= 2)
                def _():
                    wait_out(b)

                @pl.loop(0, D_MODEL, unroll=4)
                def _(d):
                    cols = colpar + d
                    vals = [
                        src[b, d, pl.ds(l0 * 16, 16)] for l0 in range(8)
                    ]
                    for l0 in range(8):
                        plsc.store_scatter(
                            dst.at[b], [rows_half + (l0 * 8), cols], vals[l0]
                        )
                fire_out(m, b)

    @pl.when(ncols >= 2)
    def _():
        wait_out(0)
        wait_out(1)

    @pl.when(wid == NW - 1)
    def _():
        pltpu.sync_copy(tail_hbm, src.at[0, pl.ds(0, 32)])
        pltpu.sync_copy(
            src.at[0, pl.ds(0, 32)], tbl_hbm.at[pl.ds(TCOLS * 64, 32)]
        )


def _repack(wt, tail2d):
    run = pl.kernel(
        _repack_body,
        out_type=jax.ShapeDtypeStruct((VOCAB // 2, 128), jnp.float32),
        mesh=plsc.VectorSubcoreMesh(**_MESH),
        scratch_types=[
            pltpu.VMEM((2, D_MODEL, 128), jnp.float32),
            pltpu.VMEM((2, D_MODEL, 128), jnp.float32),
            pltpu.SemaphoreType.DMA,
            pltpu.SemaphoreType.DMA,
        ],
        compiler_params=pltpu.CompilerParams(
            use_tc_tiling_on_sc=True, needs_layout_passes=False
        ),
    )
    return run(wt, tail2d)


UNITS_PER_W = (SEQ // 8) * 32 // NW


def _lookup_body(tokt_hbm, tbl_hbm, out_hbm, idx_v, gath, trans, gsem0, gsem1, osem):
    wid = _wid()
    iota = lax.iota(jnp.int32, 16)
    dflat = [mm * 16 + iota for mm in range(4)]
    gsems = [gsem0, gsem1]

    def wait_wb(tb):
        for i in range(8):
            pltpu.make_async_copy(
                trans.at[tb, pl.ds(0, 8), pl.ds(0, 128)],
                out_hbm.at[0, 0, 0],
                osem,
            ).wait()

    @pl.loop(0, UNITS_PER_W)
    def _(n):
        g = wid * UNITS_PER_W + n
        sblk = g // 32
        j = g - 32 * sblk
        s0 = sblk * 8
        pltpu.sync_copy(
            tokt_hbm.at[pl.ds(s0, 8), pl.ds(j * 128, 128)], idx_v
        )

        def fire_row(r8, b):
            pltpu.async_copy(tbl_hbm.at[idx_v.at[r8]], gath.at[b], gsems[b])

        def wait_row(b):
            pltpu.make_async_copy(
                tbl_hbm.at[idx_v.at[0]], gath.at[b], gsems[b]
            ).wait()

        fire_row(0, 0)
        for r8 in range(8):
            b = r8 % 2
            tb = r8 % 2
            if r8 < 7:
                fire_row(r8 + 1, 1 - b)
            wait_row(b)
            if r8 >= 2:
                wait_wb(tb)
            else:
                @pl.when(n > 0)
                def _():
                    wait_wb(tb)

            @pl.loop(0, 128, unroll=8)
            def _(t):
                colt = jnp.broadcast_to(t, (16,)).astype(jnp.int32)
                vals = [gath[b, t, pl.ds(mm * 16, 16)] for mm in range(4)]
                for mm in range(4):
                    plsc.store_scatter(
                        trans.at[tb], [dflat[mm], colt], vals[mm]
                    )
            for i in range(8):
                pltpu.async_copy(
                    trans.at[tb, pl.ds(8 * i, 8), pl.ds(0, 128)],
                    out_hbm.at[s0 + r8, i, j],
                    osem,
                )

    wait_wb(0)
    wait_wb(1)


def _lookup(tokt, tbl):
    run = pl.kernel(
        _lookup_body,
        out_type=jax.ShapeDtypeStruct((SEQ, 8, 32, 8, 128), jnp.float32),
        mesh=plsc.VectorSubcoreMesh(**_MESH),
        scratch_types=[
            pltpu.VMEM((8, 128), jnp.int32),
            pltpu.VMEM((2, 128, D_MODEL), jnp.float32),
            pltpu.VMEM((2, D_MODEL, 131), jnp.float32),
            pltpu.SemaphoreType.DMA,
            pltpu.SemaphoreType.DMA,
            pltpu.SemaphoreType.DMA,
        ],
        compiler_params=pltpu.CompilerParams(
            use_tc_tiling_on_sc=False, needs_layout_passes=False
        ),
    )
    return run(tokt, tbl)


def kernel(token_ids, weight):
    wt = weight.T
    tail2d = weight[TCOLS * 128:].reshape(32, 128)
    tbl = _repack(wt, tail2d).reshape(VOCAB, D_MODEL)
    tokt = token_ids.T.astype(jnp.int32)
    out5 = _lookup(tokt, tbl)
    return out5.transpose(2, 4, 0, 1, 3).reshape(BATCH, SEQ, D_MODEL)
